# Initial kernel scaffold; baseline (speedup 1.0000x reference)
import jax
import jax.numpy as jnp
from jax.experimental import pallas as pl


def kernel(drug_graph_embedding, edge_index, batch, global_ids, W1, b1, W2, b2):
    raise NotImplementedError("write your pallas kernel here")


# speedup vs baseline: 2.2784x; 2.2784x over previous
import functools

import jax
import jax.numpy as jnp
from jax import lax
from jax.experimental import pallas as pl
from jax.experimental.pallas import tpu as pltpu
from jax.experimental.pallas import tpu_sc as plsc

F32 = jnp.float32
I32 = jnp.int32

N = 10000
E = 320000
B = 16
P = 8
S = 1024
D = 128
H = 128

NPAD = 10240
NC = 2
NS = 16
SUB = 80
CH = 160
NSUB = CH // SUB
EPT = E // NS
NCHUNK = EPT // CH
GPC = CH // 16
DCH = 400
DNSUB = DCH // SUB
DGPC = DCH // 16
QTR = NPAD // 4
NDUMP = 256
SHROWS = QTR + NDUMP
GPAD = 2 * NPAD

_mesh = lambda: plsc.VectorSubcoreMesh(core_axis_name="c", subcore_axis_name="s")
_SC_PARAMS = pltpu.CompilerParams(needs_layout_passes=False)


def _iota16():
    return lax.iota(I32, 16)


def _tw_body(dge_ref, w1_ref, out_ref):
    x = dge_ref[0]
    m = jnp.sum(x, axis=0) * (1.0 / P)
    out_ref[0] = jnp.dot(m, w1_ref[...], preferred_element_type=F32)


def _tw_pass(dge, w1):
    return pl.pallas_call(
        _tw_body,
        grid=(B,),
        in_specs=[
            pl.BlockSpec((1, P, S, D), lambda i: (i, 0, 0, 0)),
            pl.BlockSpec((D, H), lambda i: (0, 0)),
        ],
        out_specs=pl.BlockSpec((1, S, H), lambda i: (i, 0, 0)),
        out_shape=jax.ShapeDtypeStruct((B, S, H), F32),
    )(dge, w1)


def _deg_kernel(pk1d, deg_out, dst_v, row_v, stage, deg_sh, sem):
    cid = lax.axis_index("c")
    sid = lax.axis_index("s")
    iota = _iota16()
    zeros16 = jnp.zeros((16,), F32)
    ones16 = jnp.ones((16,), F32)

    @pl.loop(0, DCH)
    def _zero_stage(r):
        stage[r, :] = zeros16

    pltpu.sync_copy(stage.at[pl.ds(0, 40)], deg_sh.at[pl.ds(sid * 40, 40)])
    plsc.subcore_barrier()

    ebase = (cid * NS + sid) * (E // (NC * NS))

    @pl.loop(0, (E // (NC * NS)) // DCH)
    def _chunk(c):
        e0 = ebase + c * DCH
        for j in range(DNSUB):
            pltpu.sync_copy(pk1d.at[pl.ds(e0 + j * SUB, SUB)], dst_v.at[j])
        for g in range(DGPC):
            r, col = g // DNSUB, (g % DNSUB) * 16
            dv = dst_v[r, pl.ds(col, 16)] & 16383
            row_v[r, pl.ds(col, 16)] = lax.shift_right_logical(dv, 4)
            plsc.store_scatter(stage, [g * 16 + iota, dv & 15], ones16)
        sdescs = []
        for j in range(DNSUB):
            sdescs.append(pltpu.async_copy(
                stage.at[pl.ds(j * SUB, SUB)],
                deg_sh.at[row_v.at[j]], sem, add=True))
        for d_ in sdescs:
            d_.wait()
        for g in range(DGPC):
            r, col = g // DNSUB, (g % DNSUB) * 16
            dv = dst_v[r, pl.ds(col, 16)] & 16383
            plsc.store_scatter(stage, [g * 16 + iota, dv & 15], zeros16)

    plsc.subcore_barrier()

    @pl.when(sid == 0)
    def _dump():
        pltpu.sync_copy(deg_sh, deg_out.at[cid])


def _deg_pass(pk1d):
    return pl.kernel(
        _deg_kernel,
        out_type=jax.ShapeDtypeStruct((NC, NPAD // 16, 16), F32),
        mesh=_mesh(), compiler_params=_SC_PARAMS,
        scratch_types=[
            pltpu.VMEM((DNSUB, SUB), I32),
            pltpu.VMEM((DNSUB, SUB), I32),
            pltpu.VMEM((DCH, 16), F32),
            pltpu.VMEM_SHARED((NPAD // 16, 16), F32),
            pltpu.SemaphoreType.DMA,
        ],
    )(pk1d)


def _rsqrt_body(degp_ref, out_ref):
    d = degp_ref[0] + degp_ref[1] + 1.0
    out_ref[...] = lax.rsqrt(d)


def _rsqrt_pass(degp):
    return pl.pallas_call(
        _rsqrt_body,
        out_shape=jax.ShapeDtypeStruct((NPAD // 16, 16), F32),
    )(degp)


def _g_kernel(tw, batch1d, gid1d, dinv1d, g_out, bv_v, gv_v, fi_v, rows_v,
              dinv_vb, sem):
    cid = lax.axis_index("c")
    sid = lax.axis_index("s")
    wid = cid * NS + sid
    n0 = wid * 320

    for j in range(4):
        pltpu.sync_copy(batch1d.at[pl.ds(n0 + j * SUB, SUB)], bv_v.at[j])
        pltpu.sync_copy(gid1d.at[pl.ds(n0 + j * SUB, SUB)], gv_v.at[j])
        pltpu.sync_copy(dinv1d.at[pl.ds(n0 + j * SUB, SUB)],
                        dinv_vb.at[pl.ds(j * SUB, SUB)])

    for g in range(20):
        r, col = g // 5, (g % 5) * 16
        bv = bv_v[r, pl.ds(col, 16)]
        gv = gv_v[r, pl.ds(col, 16)]
        fi_v[r, pl.ds(col, 16)] = jnp.minimum(bv * S + gv, B * S - 1)

    descs = [
        pltpu.async_copy(tw.at[fi_v.at[j]], rows_v.at[pl.ds(j * SUB, SUB)], sem)
        for j in range(4)
    ]
    for d_ in descs:
        d_.wait()

    dn = lax.GatherDimensionNumbers(offset_dims=(), collapsed_slice_dims=(0,),
                                    start_index_map=(0,))

    @pl.loop(0, 20)
    def _scale(g):
        dv16 = dinv_vb[pl.ds(g * 16, 16)]
        for j in range(16):
            s = lax.gather(dv16, jnp.full((16, 1), j, I32), dn, (1,),
                           mode=lax.GatherScatterMode.PROMISE_IN_BOUNDS)
            row = g * 16 + j
            for k in range(8):
                rows_v[row, pl.ds(k * 16, 16)] = (
                    rows_v[row, pl.ds(k * 16, 16)] * s)

    pltpu.sync_copy(rows_v, g_out.at[pl.ds(wid * 320, 320)])


def _g_pass(tw_flat, batch_pad, gid_pad, dinv_flat):
    return pl.kernel(
        _g_kernel,
        out_type=jax.ShapeDtypeStruct((GPAD, H), F32),
        mesh=_mesh(), compiler_params=_SC_PARAMS,
        scratch_types=[
            pltpu.VMEM((4, SUB), I32),
            pltpu.VMEM((4, SUB), I32),
            pltpu.VMEM((4, SUB), I32),
            pltpu.VMEM((320, H), F32),
            pltpu.VMEM((320,), F32),
            pltpu.SemaphoreType.DMA,
        ],
    )(tw_flat, batch_pad, gid_pad, dinv_flat)


def _edge_kernel(launch, pk1d, g_hbm, dinv_hbm, batch_hbm, out1_out, c_out,
                 pk_v, sv_v, dv_v, cv_v, rows_v, cstage, dinv_l, batch_l,
                 out1_sh, c_sh, sem):
    cid = lax.axis_index("c")
    sid = lax.axis_index("s")
    iota = _iota16()
    zeros16 = jnp.zeros((16,), F32)
    base = (2 * launch + cid) * QTR

    @pl.loop(0, NPAD // SUB)
    def _stage_nodes(j):
        pltpu.sync_copy(dinv_hbm.at[pl.ds(j * SUB, SUB)],
                        dinv_l.at[pl.ds(j * SUB, SUB)])
        pltpu.sync_copy(batch_hbm.at[pl.ds(j * SUB, SUB)],
                        batch_l.at[pl.ds(j * SUB, SUB)])

    @pl.loop(0, CH)
    def _zero_bufs(r):
        for k in range(8):
            rows_v[r, pl.ds(k * 16, 16)] = zeros16
        cstage[r, :] = zeros16

    b0z = sid * (SHROWS // NS)
    for off, ln in ((0, CH), (CH, SHROWS // NS - CH)):
        pltpu.sync_copy(rows_v.at[pl.ds(0, ln)],
                        out1_sh.at[pl.ds(b0z + off, ln)])
        pltpu.sync_copy(cstage.at[pl.ds(0, ln)],
                        c_sh.at[pl.ds(b0z + off, ln)])
    plsc.subcore_barrier()

    ebase = sid * EPT

    @pl.loop(0, NCHUNK)
    def _chunk(c):
        e0 = ebase + c * CH
        for j in range(NSUB):
            pltpu.sync_copy(pk1d.at[pl.ds(e0 + j * SUB, SUB)], pk_v.at[j])
        for g in range(GPC):
            r, col = g // NSUB, (g % NSUB) * 16
            pv = pk_v[r, pl.ds(col, 16)]
            sv = lax.shift_right_logical(pv, 14)
            dv = pv & 16383
            sv_v[r, pl.ds(col, 16)] = sv
            dloc = dv - base
            din = (dloc >= 0) & (dloc < QTR)
            dv_v[r, pl.ds(col, 16)] = jnp.where(
                din, dloc, QTR + (dv & (NDUMP - 1)))
        descs = [
            pltpu.async_copy(g_hbm.at[sv_v.at[j]],
                             rows_v.at[pl.ds(j * SUB, SUB)], sem)
            for j in range(NSUB)
        ]
        for g in range(GPC):
            r, col = g // NSUB, (g % NSUB) * 16
            sv = sv_v[r, pl.ds(col, 16)]
            pv = pk_v[r, pl.ds(col, 16)]
            dv = pv & 16383
            dis = plsc.load_gather(dinv_l, [sv])
            did = plsc.load_gather(dinv_l, [dv])
            bv = plsc.load_gather(batch_l, [dv])
            plsc.store_scatter(cstage, [g * 16 + iota, bv], dis * did)
            sloc = sv - base
            sin = (sloc >= 0) & (sloc < QTR)
            cv_v[r, pl.ds(col, 16)] = jnp.where(
                sin, sloc, QTR + (sv & (NDUMP - 1)))
        for d_ in descs:
            d_.wait()
        sdescs = []
        for j in range(NSUB):
            sdescs.append(pltpu.async_copy(
                rows_v.at[pl.ds(j * SUB, SUB)],
                out1_sh.at[dv_v.at[j]], sem, add=True))
            sdescs.append(pltpu.async_copy(
                cstage.at[pl.ds(j * SUB, SUB)],
                c_sh.at[cv_v.at[j]], sem, add=True))
        for d_ in sdescs:
            d_.wait()
        for g in range(GPC):
            r, col = g // NSUB, (g % NSUB) * 16
            pv = pk_v[r, pl.ds(col, 16)]
            dv = pv & 16383
            bv = plsc.load_gather(batch_l, [dv])
            plsc.store_scatter(cstage, [g * 16 + iota, bv], zeros16)

    nb0 = base + sid * (QTR // NS)
    for g in range(10):
        dv16 = dinv_l[pl.ds(nb0 + g * 16, 16)]
        bv = batch_l[pl.ds(nb0 + g * 16, 16)]
        plsc.store_scatter(cstage, [g * 16 + iota, bv],
                           dv16 * dv16, mask=bv < B)
        sv_v[g // NSUB, pl.ds((g % NSUB) * 16, 16)] = (
            sid * (QTR // NS) + g * 16 + iota)
    for j in range(2):
        pltpu.sync_copy(cstage.at[pl.ds(j * SUB, SUB)],
                        c_sh.at[sv_v.at[j]], add=True)

    plsc.subcore_barrier()

    b0 = sid * (QTR // NS)
    pltpu.sync_copy(out1_sh.at[pl.ds(b0, QTR // NS)],
                    out1_out.at[cid, pl.ds(b0, QTR // NS)])
    pltpu.sync_copy(c_sh.at[pl.ds(b0, QTR // NS)],
                    c_out.at[cid, pl.ds(b0, QTR // NS)])


def _edge_pass(pk1d, g_flat, dinv_flat, batch_pad, launch):
    return pl.kernel(
        functools.partial(_edge_kernel, launch),
        out_type=[
            jax.ShapeDtypeStruct((NC, QTR, H), F32),
            jax.ShapeDtypeStruct((NC, QTR, B), F32),
        ],
        mesh=_mesh(), compiler_params=_SC_PARAMS,
        scratch_types=[
            pltpu.VMEM((NSUB, SUB), I32),
            pltpu.VMEM((NSUB, SUB), I32),
            pltpu.VMEM((NSUB, SUB), I32),
            pltpu.VMEM((NSUB, SUB), I32),
            pltpu.VMEM((CH, H), F32),
            pltpu.VMEM((CH, B), F32),
            pltpu.VMEM((NPAD,), F32),
            pltpu.VMEM((NPAD,), I32),
            pltpu.VMEM_SHARED((SHROWS, H), F32),
            pltpu.VMEM_SHARED((SHROWS, B), F32),
            pltpu.SemaphoreType.DMA,
        ],
    )(pk1d, g_flat, dinv_flat, batch_pad)


_NBLK = 8
_BLK = NPAD // _NBLK


def _final_body(p, g, cmat, dinv, batchc, w2, b1, b2, out, pooled,
                counts):
    i = pl.program_id(0)

    @pl.when(i == 0)
    def _init():
        pooled[...] = jnp.zeros((B, H), F32)
        counts[...] = jnp.zeros((B, H), F32)

    h1 = jnp.maximum(dinv[...] * (p[...] + g[...]) + b1[...], 0.0)
    csum = cmat[...]
    pooled[...] += lax.dot_general(csum, h1, (((0,), (0,)), ((), ())),
                                   preferred_element_type=F32)
    onehot = (batchc[...] == lax.broadcasted_iota(I32, (1, B), 1)).astype(F32)
    counts[...] += lax.dot_general(onehot, jnp.ones((_BLK, H), F32),
                                   (((0,), (0,)), ((), ())),
                                   preferred_element_type=F32)

    @pl.when(i == _NBLK - 1)
    def _fin():
        out[...] = (jnp.dot(pooled[...], w2[...], preferred_element_type=F32)
                    / counts[...]) + b2[...]


def _final_pass(p, g, cmat, dinv_col, batch_col, w2, b1, b2):
    return pl.pallas_call(
        _final_body,
        grid=(_NBLK,),
        in_specs=[
            pl.BlockSpec((_BLK, H), lambda i: (i, 0)),
            pl.BlockSpec((_BLK, H), lambda i: (i, 0)),
            pl.BlockSpec((_BLK, B), lambda i: (i, 0)),
            pl.BlockSpec((_BLK, 1), lambda i: (i, 0)),
            pl.BlockSpec((_BLK, 1), lambda i: (i, 0)),
            pl.BlockSpec((H, H), lambda i: (0, 0)),
            pl.BlockSpec((1, H), lambda i: (0, 0)),
            pl.BlockSpec((1, H), lambda i: (0, 0)),
        ],
        out_specs=pl.BlockSpec((B, H), lambda i: (0, 0)),
        out_shape=jax.ShapeDtypeStruct((B, H), F32),
        scratch_shapes=[
            pltpu.VMEM((B, H), F32),
            pltpu.VMEM((B, H), F32),
        ],
    )(p, g, cmat, dinv_col, batch_col, w2, b1, b2)


def kernel(drug_graph_embedding, edge_index, batch, global_ids, W1, b1, W2,
           b2):
    tw = _tw_pass(drug_graph_embedding, W1).reshape(B * S, H)

    pk1d = edge_index[0] * 16384 + edge_index[1]
    batch_pad = jnp.concatenate([batch, jnp.full((NPAD - N,), B, I32)])
    gid_pad = jnp.concatenate([global_ids, jnp.zeros((NPAD - N,), I32)])

    degp = _deg_pass(pk1d)
    dinv2d = _rsqrt_pass(degp)
    dinv_flat = dinv2d.reshape(NPAD)

    g = _g_pass(tw, batch_pad, gid_pad, dinv_flat)

    o0, c0 = _edge_pass(pk1d, g, dinv_flat, batch_pad, 0)
    o1, c1 = _edge_pass(pk1d, g, dinv_flat, batch_pad, 1)
    out1p = jnp.concatenate([o0.reshape(NPAD // 2, H),
                             o1.reshape(NPAD // 2, H)])
    cp = jnp.concatenate([c0.reshape(NPAD // 2, B),
                          c1.reshape(NPAD // 2, B)])

    return _final_pass(out1p, g[:NPAD], cp,
                       dinv_flat.reshape(NPAD, 1),
                       batch_pad.reshape(NPAD, 1), W2,
                       b1.reshape(1, H), b2.reshape(1, H))
